# baseline (speedup 1.0000x reference)
import functools

import jax
import jax.numpy as jnp
from jax import lax
from jax.experimental import pallas as pl
from jax.experimental.pallas import tpu as pltpu
from jax.experimental.pallas import tpu_sc as plsc

F32 = jnp.float32
I32 = jnp.int32

E = 8
ENC = 256
WID = 256
NXF = 10
NDF = 4
T = 256

SC_CORES = 2
SC_SUBCORES = 16
NWORK = SC_CORES * SC_SUBCORES
CHUNK = 128


def _pe_feats(x, degree):
    feats = [x]
    for d in range(degree):
        feats.append(jnp.sin((2.0 ** d) * x))
    for d in range(degree):
        feats.append(jnp.cos((2.0 ** d) * x))
    return jnp.concatenate(feats, axis=-1)


def _gating_body(temp_ref, xyz_ref, wenc_ref, benc_ref, wg_ref, bg_ref,
                 y_ref, gates_ref, onehot_ref, gtop_ref, counts_ref, gsum_ref):
    i = pl.program_id(0)
    x = xyz_ref[...]
    px = _pe_feats(x, NXF)
    bA = px.shape[0]
    px = jnp.concatenate([px, jnp.zeros((bA, 1), F32)], axis=1)
    y = jnp.dot(px, wenc_ref[...], preferred_element_type=F32) + benc_ref[...]
    y_ref[...] = y
    logits = jnp.dot(y, wg_ref[...], preferred_element_type=F32) + bg_ref[...]
    lt = logits / temp_ref[0, 0]
    m = jnp.max(lt, axis=1, keepdims=True)
    ex = jnp.exp(lt - m)
    g = ex / jnp.sum(ex, axis=1, keepdims=True)
    gates_ref[...] = g
    li = lax.broadcasted_iota(I32, g.shape, 1)
    gm = jnp.max(g, axis=1, keepdims=True)
    am = jnp.min(jnp.where(g == gm, li, E), axis=1, keepdims=True)
    oh = (li == am).astype(F32)
    onehot_ref[...] = oh
    gtop_ref[...] = gm

    @pl.when(i == 0)
    def _():
        counts_ref[...] = jnp.zeros_like(counts_ref)
        gsum_ref[...] = jnp.zeros_like(gsum_ref)

    counts_ref[...] += jnp.sum(oh, axis=0, keepdims=True)
    gsum_ref[...] += jnp.sum(g, axis=0, keepdims=True)


def _gating(xyz2, temp11, wenc64, b_enc, W_g, b_g, n):
    ba = 512
    grid = (n // ba,)
    return pl.pallas_call(
        _gating_body,
        grid=grid,
        in_specs=[
            pl.BlockSpec(memory_space=pltpu.SMEM),
            pl.BlockSpec((ba, 3), lambda i: (i, 0)),
            pl.BlockSpec((64, ENC), lambda i: (0, 0)),
            pl.BlockSpec((1, ENC), lambda i: (0, 0)),
            pl.BlockSpec((ENC, E), lambda i: (0, 0)),
            pl.BlockSpec((1, E), lambda i: (0, 0)),
        ],
        out_specs=[
            pl.BlockSpec((ba, ENC), lambda i: (i, 0)),
            pl.BlockSpec((ba, E), lambda i: (i, 0)),
            pl.BlockSpec((ba, E), lambda i: (i, 0)),
            pl.BlockSpec((ba, 1), lambda i: (i, 0)),
            pl.BlockSpec((1, E), lambda i: (0, 0)),
            pl.BlockSpec((1, E), lambda i: (0, 0)),
        ],
        out_shape=[
            jax.ShapeDtypeStruct((n, ENC), F32),
            jax.ShapeDtypeStruct((n, E), F32),
            jax.ShapeDtypeStruct((n, E), F32),
            jax.ShapeDtypeStruct((n, 1), F32),
            jax.ShapeDtypeStruct((1, E), F32),
            jax.ShapeDtypeStruct((1, E), F32),
        ],
    )(temp11, xyz2, wenc64, b_enc.reshape(1, ENC), W_g, b_g.reshape(1, E))


def _dest_body(onehot_ref, starts_ref, dest_ref, carry_ref):
    i = pl.program_id(0)

    @pl.when(i == 0)
    def _():
        carry_ref[...] = jnp.zeros_like(carry_ref)

    oh = onehot_ref[...]
    tb = oh.shape[0]
    r = lax.broadcasted_iota(I32, (tb, tb), 0)
    c = lax.broadcasted_iota(I32, (tb, tb), 1)
    ltri = (r > c).astype(F32)
    ranks = jnp.dot(ltri, oh, preferred_element_type=F32)
    base = starts_ref[...] + carry_ref[...]
    destf = jnp.sum(oh * (base + ranks), axis=1, keepdims=True)
    dest_ref[...] = destf.astype(I32)
    carry_ref[...] += jnp.sum(oh, axis=0, keepdims=True)


def _dest(onehot, starts18, n):
    tb = 512
    return pl.pallas_call(
        _dest_body,
        grid=(n // tb,),
        in_specs=[
            pl.BlockSpec((tb, E), lambda i: (i, 0)),
            pl.BlockSpec((1, E), lambda i: (0, 0)),
        ],
        out_specs=pl.BlockSpec((tb, 1), lambda i: (i, 0)),
        out_shape=jax.ShapeDtypeStruct((n, 1), I32),
        scratch_shapes=[pltpu.VMEM((1, E), F32)],
    )(onehot, starts18)


def _dispatch_scatter(y, dest3, npad):
    n = y.shape[0]
    per_w = n // NWORK
    nchunks = per_w // CHUNK
    mesh = plsc.VectorSubcoreMesh(core_axis_name="c", subcore_axis_name="s")

    @functools.partial(
        pl.kernel,
        mesh=mesh,
        out_type=jax.ShapeDtypeStruct((npad, ENC), F32),
        scratch_types=[
            pltpu.VMEM((nchunks, CHUNK), I32),
            pltpu.VMEM((CHUNK, ENC), F32),
            pltpu.SemaphoreType.DMA,
        ],
    )
    def k(y_hbm, dest_hbm, ys_hbm, idx_v, row_v, sem):
        wid = lax.axis_index("s") * SC_CORES + lax.axis_index("c")
        pltpu.sync_copy(dest_hbm.at[wid], idx_v)
        base = wid * per_w
        for j in range(nchunks):
            pltpu.sync_copy(y_hbm.at[pl.ds(base + j * CHUNK, CHUNK)], row_v)
            pltpu.async_copy(row_v, ys_hbm.at[idx_v.at[j]], sem).wait()

    return k(y, dest3)


def _expert_body(eid_ref, ys_ref, w1_ref, b1_ref, w2_ref, b2_ref,
                 w3_ref, b3_ref, w4_ref, b4_ref, out_ref):
    a = ys_ref[...]
    h = jnp.maximum(jnp.dot(a, w1_ref[0], preferred_element_type=F32) + b1_ref[0], 0.0)
    h = jnp.maximum(jnp.dot(h, w2_ref[0], preferred_element_type=F32) + b2_ref[0], 0.0)
    h = jnp.maximum(jnp.dot(h, w3_ref[0], preferred_element_type=F32) + b3_ref[0], 0.0)
    out_ref[...] = jnp.dot(h, w4_ref[0], preferred_element_type=F32) + b4_ref[0]


def _experts(tile_eid, ys, We1, be1, We2, be2, We3, be3, We4, be4, npad):
    nt = npad // T
    wspec = pl.BlockSpec((1, ENC, WID), lambda t, eid: (eid[t], 0, 0))
    bspec = pl.BlockSpec((1, 1, WID), lambda t, eid: (eid[t], 0, 0))
    grid_spec = pltpu.PrefetchScalarGridSpec(
        num_scalar_prefetch=1,
        grid=(nt,),
        in_specs=[
            pl.BlockSpec((T, ENC), lambda t, eid: (t, 0)),
            wspec, bspec, wspec, bspec, wspec, bspec, wspec, bspec,
        ],
        out_specs=pl.BlockSpec((T, WID), lambda t, eid: (t, 0)),
    )
    return pl.pallas_call(
        _expert_body,
        grid_spec=grid_spec,
        out_shape=jax.ShapeDtypeStruct((npad, WID), F32),
    )(tile_eid, ys,
      We1, be1.reshape(E, 1, WID), We2, be2.reshape(E, 1, WID),
      We3, be3.reshape(E, 1, WID), We4, be4.reshape(E, 1, WID))


def _combine_gather(hs, dest3, n):
    per_w = n // NWORK
    nchunks = per_w // CHUNK
    mesh = plsc.VectorSubcoreMesh(core_axis_name="c", subcore_axis_name="s")

    @functools.partial(
        pl.kernel,
        mesh=mesh,
        out_type=jax.ShapeDtypeStruct((n, WID), F32),
        scratch_types=[
            pltpu.VMEM((nchunks, CHUNK), I32),
            pltpu.VMEM((CHUNK, WID), F32),
            pltpu.SemaphoreType.DMA,
        ],
    )
    def k(hs_hbm, dest_hbm, out_hbm, idx_v, row_v, sem):
        wid = lax.axis_index("s") * SC_CORES + lax.axis_index("c")
        pltpu.sync_copy(dest_hbm.at[wid], idx_v)
        base = wid * per_w
        for j in range(nchunks):
            pltpu.async_copy(hs_hbm.at[idx_v.at[j]], row_v, sem).wait()
            pltpu.sync_copy(row_v, out_hbm.at[pl.ds(base + j * CHUNK, CHUNK)])

    return k(hs, dest3)


def _head_body(bsig_ref, hraw_ref, gtop_ref, vdir_ref, wsig_ref, wr1a_ref,
               wr1b_ref, br1_ref, wr2_ref, br2_ref,
               sig_ref, rgb_ref, ssum_ref):
    i = pl.program_id(0)
    so = hraw_ref[...] * gtop_ref[...]
    z = jnp.sum(so * wsig_ref[...], axis=1, keepdims=True) + bsig_ref[0, 0]
    sig = jnp.maximum(z, 0.0) + jnp.log(1.0 + jnp.exp(-jnp.abs(z)))
    sig_ref[...] = sig
    v = vdir_ref[...]
    vd = _pe_feats(v, NDF)
    bf = vd.shape[0]
    vd = jnp.concatenate([vd, jnp.zeros((bf, 5), F32)], axis=1)
    hr = (jnp.dot(so, wr1a_ref[...], preferred_element_type=F32)
          + jnp.dot(vd, wr1b_ref[...], preferred_element_type=F32)
          + br1_ref[...])
    hr = jnp.maximum(hr, 0.0)
    t = jnp.dot(hr, wr2_ref[...], preferred_element_type=F32) + br2_ref[...]
    rgb_ref[...] = 1.0 / (1.0 + jnp.exp(-t))

    @pl.when(i == 0)
    def _():
        ssum_ref[...] = jnp.zeros_like(ssum_ref)

    ssum_ref[...] += jnp.sum(sig, axis=0, keepdims=True)


def _heads(hraw, gtop, vdir2, wsig_row, bsig11, wr1a, wr1b32, b_r1,
           wr2p, br2p, n):
    bf = 1024
    return pl.pallas_call(
        _head_body,
        grid=(n // bf,),
        in_specs=[
            pl.BlockSpec(memory_space=pltpu.SMEM),
            pl.BlockSpec((bf, ENC), lambda i: (i, 0)),
            pl.BlockSpec((bf, 1), lambda i: (i, 0)),
            pl.BlockSpec((bf, 3), lambda i: (i, 0)),
            pl.BlockSpec((1, ENC), lambda i: (0, 0)),
            pl.BlockSpec((ENC, 128), lambda i: (0, 0)),
            pl.BlockSpec((32, 128), lambda i: (0, 0)),
            pl.BlockSpec((1, 128), lambda i: (0, 0)),
            pl.BlockSpec((128, 128), lambda i: (0, 0)),
            pl.BlockSpec((1, 128), lambda i: (0, 0)),
        ],
        out_specs=[
            pl.BlockSpec((bf, 1), lambda i: (i, 0)),
            pl.BlockSpec((bf, 128), lambda i: (i, 0)),
            pl.BlockSpec((1, 1), lambda i: (0, 0)),
        ],
        out_shape=[
            jax.ShapeDtypeStruct((n, 1), F32),
            jax.ShapeDtypeStruct((n, 128), F32),
            jax.ShapeDtypeStruct((1, 1), F32),
        ],
    )(bsig11, hraw, gtop, vdir2, wsig_row, wr1a, wr1b32,
      b_r1.reshape(1, 128), wr2p, br2p)


def kernel(xyz, viewdir, shape_latent, texture_latent, temperature,
           W_enc, b_enc, W_g, b_g,
           We1, be1, We2, be2, We3, be3, We4, be4,
           W_sig, b_sig, W_r1, b_r1, W_r2, b_r2):
    nrays, nsamples, _ = xyz.shape
    n = nrays * nsamples
    npad = (n // T + E) * T

    xyz2 = xyz.reshape(n, 3)
    vdir2 = viewdir.reshape(n, 3)
    temp11 = temperature.reshape(1, 1)
    d_xyz = W_enc.shape[0]
    wenc64 = jnp.concatenate([W_enc, jnp.zeros((64 - d_xyz, ENC), F32)], axis=0)

    y, gates, onehot, gtop, counts, gsum = _gating(
        xyz2, temp11, wenc64, b_enc, W_g, b_g, n)

    cnt = counts.reshape(E)
    tile_cnt = jnp.ceil(cnt / T).astype(I32)
    tile_start = jnp.concatenate(
        [jnp.zeros((1,), I32), jnp.cumsum(tile_cnt)[:-1]])
    starts18 = (tile_start * T).astype(F32).reshape(1, E)
    nt = npad // T
    cum = jnp.cumsum(tile_cnt)
    tile_eid = jnp.minimum(
        jnp.searchsorted(cum, jnp.arange(nt, dtype=I32), side="right"),
        E - 1).astype(I32)

    dest = _dest(onehot, starts18, n)
    dest3 = dest.reshape(NWORK, (n // NWORK) // CHUNK, CHUNK)

    ys = _dispatch_scatter(y, dest3, npad)
    hs = _experts(tile_eid, ys, We1, be1, We2, be2, We3, be3, We4, be4, npad)
    hraw = _combine_gather(hs, dest3, n)

    wsig_row = W_sig.reshape(1, ENC)
    bsig11 = b_sig.reshape(1, 1)
    d_dir = W_r1.shape[0] - ENC
    wr1a = W_r1[:ENC]
    wr1b32 = jnp.concatenate(
        [W_r1[ENC:], jnp.zeros((32 - d_dir, 128), F32)], axis=0)
    wr2p = jnp.concatenate([W_r2, jnp.zeros((128, 125), F32)], axis=1)
    br2p = jnp.concatenate([b_r2, jnp.zeros((125,), F32)]).reshape(1, 128)

    sig, rgbp, ssum = _heads(hraw, gtop, vdir2, wsig_row, bsig11,
                             wr1a, wr1b32, b_r1, wr2p, br2p, n)

    sigmas = sig.reshape(nrays, nsamples, 1)
    rgbs = rgbp[:, :3].reshape(nrays, nsamples, 3)
    gates_soft_o = gates.reshape(nrays, nsamples, E)
    gates_hard_o = onehot.reshape(nrays, nsamples, E)
    mean_sigma = (ssum / n).reshape(1)
    num_pts = cnt
    aux_loss = E * jnp.sum((cnt / n) * (gsum.reshape(E) / n))
    return (sigmas, rgbs, gates_soft_o, gates_hard_o,
            mean_sigma, num_pts, aux_loss)


# speedup vs baseline: 2.4535x; 2.4535x over previous
import functools

import jax
import jax.numpy as jnp
import numpy as np
from jax import lax
from jax.experimental import pallas as pl
from jax.experimental.pallas import tpu as pltpu
from jax.experimental.pallas import tpu_sc as plsc

F32 = jnp.float32
I32 = jnp.int32

E = 8
ENC = 256
WID = 256
NXF = 10
NDF = 4
T = 256

SC_CORES = 2
SC_SUBCORES = 16
NWORK = SC_CORES * SC_SUBCORES
CHUNK = 128


def _pe_matrix(degree, width):
    m = np.zeros((3, width), np.float32)
    for c in range(3):
        m[c, c] = 1.0
    for d in range(degree):
        for c in range(3):
            m[c, 3 + 3 * d + c] = 2.0 ** d
            m[c, 3 + 3 * degree + 3 * d + c] = 2.0 ** d
    return jnp.asarray(m)


def _pe_apply(x, mat, degree):
    t = jnp.dot(x, mat, preferred_element_type=F32)
    li = lax.broadcasted_iota(I32, t.shape, 1)
    return jnp.where(li < 3, t,
                     jnp.where(li < 3 + 3 * degree, jnp.sin(t), jnp.cos(t)))


def _gating_body(temp_ref, xyz_ref, pemat_ref, wenc_ref, benc_ref, wg_ref,
                 bg_ref,
                 y_ref, gates_ref, onehot_ref, gtop_ref, counts_ref, gsum_ref):
    i = pl.program_id(0)
    x = xyz_ref[...]
    px = _pe_apply(x, pemat_ref[...], NXF)
    y = jnp.dot(px, wenc_ref[...], preferred_element_type=F32) + benc_ref[...]
    y_ref[...] = y
    logits = jnp.dot(y, wg_ref[...], preferred_element_type=F32) + bg_ref[...]
    lt = logits / temp_ref[0, 0]
    m = jnp.max(lt, axis=1, keepdims=True)
    ex = jnp.exp(lt - m)
    g = ex / jnp.sum(ex, axis=1, keepdims=True)
    gates_ref[...] = g
    li = lax.broadcasted_iota(I32, g.shape, 1)
    gm = jnp.max(g, axis=1, keepdims=True)
    am = jnp.min(jnp.where(g == gm, li, E), axis=1, keepdims=True)
    oh = (li == am).astype(F32)
    onehot_ref[...] = oh
    gtop_ref[...] = gm

    @pl.when(i == 0)
    def _():
        counts_ref[...] = jnp.zeros_like(counts_ref)
        gsum_ref[...] = jnp.zeros_like(gsum_ref)

    counts_ref[...] += jnp.sum(oh, axis=0, keepdims=True)
    gsum_ref[...] += jnp.sum(g, axis=0, keepdims=True)


def _gating(xyz2, temp11, wenc64, b_enc, W_g, b_g, n):
    ba = 1024
    grid = (n // ba,)
    return pl.pallas_call(
        _gating_body,
        grid=grid,
        in_specs=[
            pl.BlockSpec(memory_space=pltpu.SMEM),
            pl.BlockSpec((ba, 3), lambda i: (i, 0)),
            pl.BlockSpec((3, 64), lambda i: (0, 0)),
            pl.BlockSpec((64, ENC), lambda i: (0, 0)),
            pl.BlockSpec((1, ENC), lambda i: (0, 0)),
            pl.BlockSpec((ENC, E), lambda i: (0, 0)),
            pl.BlockSpec((1, E), lambda i: (0, 0)),
        ],
        out_specs=[
            pl.BlockSpec((ba, ENC), lambda i: (i, 0)),
            pl.BlockSpec((ba, E), lambda i: (i, 0)),
            pl.BlockSpec((ba, E), lambda i: (i, 0)),
            pl.BlockSpec((ba, 1), lambda i: (i, 0)),
            pl.BlockSpec((1, E), lambda i: (0, 0)),
            pl.BlockSpec((1, E), lambda i: (0, 0)),
        ],
        out_shape=[
            jax.ShapeDtypeStruct((n, ENC), F32),
            jax.ShapeDtypeStruct((n, E), F32),
            jax.ShapeDtypeStruct((n, E), F32),
            jax.ShapeDtypeStruct((n, 1), F32),
            jax.ShapeDtypeStruct((1, E), F32),
            jax.ShapeDtypeStruct((1, E), F32),
        ],
    )(temp11, xyz2, _pe_matrix(NXF, 64), wenc64,
      b_enc.reshape(1, ENC), W_g, b_g.reshape(1, E))


def _dest_body(onehot_ref, starts_ref, dest_ref, carry_ref):
    i = pl.program_id(0)

    @pl.when(i == 0)
    def _():
        carry_ref[...] = jnp.zeros_like(carry_ref)

    oh = onehot_ref[...]
    tb = oh.shape[0]
    r = lax.broadcasted_iota(I32, (tb, tb), 0)
    c = lax.broadcasted_iota(I32, (tb, tb), 1)
    ltri = (r > c).astype(F32)
    ranks = jnp.dot(ltri, oh, preferred_element_type=F32)
    base = starts_ref[...] + carry_ref[...]
    destf = jnp.sum(oh * (base + ranks), axis=1, keepdims=True)
    dest_ref[...] = destf.astype(I32)
    carry_ref[...] += jnp.sum(oh, axis=0, keepdims=True)


def _dest(onehot, starts18, n):
    tb = 512
    return pl.pallas_call(
        _dest_body,
        grid=(n // tb,),
        in_specs=[
            pl.BlockSpec((tb, E), lambda i: (i, 0)),
            pl.BlockSpec((1, E), lambda i: (0, 0)),
        ],
        out_specs=pl.BlockSpec((tb, 1), lambda i: (i, 0)),
        out_shape=jax.ShapeDtypeStruct((n, 1), I32),
        scratch_shapes=[pltpu.VMEM((1, E), F32)],
    )(onehot, starts18)


def _dispatch_scatter(y, dest3, npad):
    n = y.shape[0]
    per_w = n // NWORK
    nchunks = per_w // CHUNK
    mesh = plsc.VectorSubcoreMesh(core_axis_name="c", subcore_axis_name="s")

    @functools.partial(
        pl.kernel,
        mesh=mesh,
        out_type=jax.ShapeDtypeStruct((npad, ENC), F32),
        scratch_types=[
            pltpu.VMEM((nchunks, CHUNK), I32),
            pltpu.VMEM((CHUNK, ENC), F32),
            pltpu.SemaphoreType.DMA,
        ],
    )
    def k(y_hbm, dest_hbm, ys_hbm, idx_v, row_v, sem):
        wid = lax.axis_index("s") * SC_CORES + lax.axis_index("c")
        pltpu.sync_copy(dest_hbm.at[wid], idx_v)
        base = wid * per_w
        for j in range(nchunks):
            pltpu.sync_copy(y_hbm.at[pl.ds(base + j * CHUNK, CHUNK)], row_v)
            pltpu.async_copy(row_v, ys_hbm.at[idx_v.at[j]], sem).wait()

    return k(y, dest3)


def _expert_body(eid_ref, ys_ref, w1_ref, b1_ref, w2_ref, b2_ref,
                 w3_ref, b3_ref, w4_ref, b4_ref, out_ref):
    a = ys_ref[...]
    h = jnp.maximum(jnp.dot(a, w1_ref[0], preferred_element_type=F32) + b1_ref[0], 0.0)
    h = jnp.maximum(jnp.dot(h, w2_ref[0], preferred_element_type=F32) + b2_ref[0], 0.0)
    h = jnp.maximum(jnp.dot(h, w3_ref[0], preferred_element_type=F32) + b3_ref[0], 0.0)
    out_ref[...] = jnp.dot(h, w4_ref[0], preferred_element_type=F32) + b4_ref[0]


def _experts(tile_eid, ys, We1, be1, We2, be2, We3, be3, We4, be4, npad):
    nt = npad // T
    wspec = pl.BlockSpec((1, ENC, WID), lambda t, eid: (eid[t], 0, 0))
    bspec = pl.BlockSpec((1, 1, WID), lambda t, eid: (eid[t], 0, 0))
    grid_spec = pltpu.PrefetchScalarGridSpec(
        num_scalar_prefetch=1,
        grid=(nt,),
        in_specs=[
            pl.BlockSpec((T, ENC), lambda t, eid: (t, 0)),
            wspec, bspec, wspec, bspec, wspec, bspec, wspec, bspec,
        ],
        out_specs=pl.BlockSpec((T, WID), lambda t, eid: (t, 0)),
    )
    return pl.pallas_call(
        _expert_body,
        grid_spec=grid_spec,
        out_shape=jax.ShapeDtypeStruct((npad, WID), F32),
    )(tile_eid, ys,
      We1, be1.reshape(E, 1, WID), We2, be2.reshape(E, 1, WID),
      We3, be3.reshape(E, 1, WID), We4, be4.reshape(E, 1, WID))


def _combine_gather(hs, dest3, n):
    per_w = n // NWORK
    nchunks = per_w // CHUNK
    mesh = plsc.VectorSubcoreMesh(core_axis_name="c", subcore_axis_name="s")

    @functools.partial(
        pl.kernel,
        mesh=mesh,
        out_type=jax.ShapeDtypeStruct((n, WID), F32),
        scratch_types=[
            pltpu.VMEM((nchunks, CHUNK), I32),
            pltpu.VMEM((CHUNK, WID), F32),
            pltpu.SemaphoreType.DMA,
        ],
    )
    def k(hs_hbm, dest_hbm, out_hbm, idx_v, row_v, sem):
        wid = lax.axis_index("s") * SC_CORES + lax.axis_index("c")
        pltpu.sync_copy(dest_hbm.at[wid], idx_v)
        base = wid * per_w
        for j in range(nchunks):
            pltpu.async_copy(hs_hbm.at[idx_v.at[j]], row_v, sem).wait()
            pltpu.sync_copy(row_v, out_hbm.at[pl.ds(base + j * CHUNK, CHUNK)])

    return k(hs, dest3)


def _head_body(bsig_ref, hraw_ref, gtop_ref, vdir_ref, pemat_ref, wsig_ref,
               wr1a_ref, wr1b_ref, br1_ref, wr2_ref, br2_ref,
               sig_ref, rgb_ref, ssum_ref):
    i = pl.program_id(0)
    so = hraw_ref[...] * gtop_ref[...]
    z = jnp.sum(so * wsig_ref[...], axis=1, keepdims=True) + bsig_ref[0, 0]
    sig = jnp.maximum(z, 0.0) + jnp.log(1.0 + jnp.exp(-jnp.abs(z)))
    sig_ref[...] = sig
    v = vdir_ref[...]
    vd = _pe_apply(v, pemat_ref[...], NDF)
    hr = (jnp.dot(so, wr1a_ref[...], preferred_element_type=F32)
          + jnp.dot(vd, wr1b_ref[...], preferred_element_type=F32)
          + br1_ref[...])
    hr = jnp.maximum(hr, 0.0)
    t = jnp.dot(hr, wr2_ref[...], preferred_element_type=F32) + br2_ref[...]
    rgb_ref[...] = 1.0 / (1.0 + jnp.exp(-t))

    @pl.when(i == 0)
    def _():
        ssum_ref[...] = jnp.zeros_like(ssum_ref)

    ssum_ref[...] += jnp.sum(sig, axis=0, keepdims=True)


def _heads(hraw, gtop, vdir2, wsig_row, bsig11, wr1a, wr1b32, b_r1,
           wr2p, br2p, n):
    bf = 1024
    return pl.pallas_call(
        _head_body,
        grid=(n // bf,),
        in_specs=[
            pl.BlockSpec(memory_space=pltpu.SMEM),
            pl.BlockSpec((bf, ENC), lambda i: (i, 0)),
            pl.BlockSpec((bf, 1), lambda i: (i, 0)),
            pl.BlockSpec((bf, 3), lambda i: (i, 0)),
            pl.BlockSpec((3, 32), lambda i: (0, 0)),
            pl.BlockSpec((1, ENC), lambda i: (0, 0)),
            pl.BlockSpec((ENC, 128), lambda i: (0, 0)),
            pl.BlockSpec((32, 128), lambda i: (0, 0)),
            pl.BlockSpec((1, 128), lambda i: (0, 0)),
            pl.BlockSpec((128, 128), lambda i: (0, 0)),
            pl.BlockSpec((1, 128), lambda i: (0, 0)),
        ],
        out_specs=[
            pl.BlockSpec((bf, 1), lambda i: (i, 0)),
            pl.BlockSpec((bf, 128), lambda i: (i, 0)),
            pl.BlockSpec((1, 1), lambda i: (0, 0)),
        ],
        out_shape=[
            jax.ShapeDtypeStruct((n, 1), F32),
            jax.ShapeDtypeStruct((n, 128), F32),
            jax.ShapeDtypeStruct((1, 1), F32),
        ],
    )(bsig11, hraw, gtop, vdir2, _pe_matrix(NDF, 32), wsig_row, wr1a, wr1b32,
      b_r1.reshape(1, 128), wr2p, br2p)


def kernel(xyz, viewdir, shape_latent, texture_latent, temperature,
           W_enc, b_enc, W_g, b_g,
           We1, be1, We2, be2, We3, be3, We4, be4,
           W_sig, b_sig, W_r1, b_r1, W_r2, b_r2):
    nrays, nsamples, _ = xyz.shape
    n = nrays * nsamples
    npad = (n // T + E) * T

    xyz2 = xyz.reshape(n, 3)
    vdir2 = viewdir.reshape(n, 3)
    temp11 = temperature.reshape(1, 1)
    d_xyz = W_enc.shape[0]
    wenc64 = jnp.concatenate([W_enc, jnp.zeros((64 - d_xyz, ENC), F32)], axis=0)

    y, gates, onehot, gtop, counts, gsum = _gating(
        xyz2, temp11, wenc64, b_enc, W_g, b_g, n)

    cnt = counts.reshape(E)
    tile_cnt = jnp.ceil(cnt / T).astype(I32)
    tile_start = jnp.concatenate(
        [jnp.zeros((1,), I32), jnp.cumsum(tile_cnt)[:-1]])
    starts18 = (tile_start * T).astype(F32).reshape(1, E)
    nt = npad // T
    cum = jnp.cumsum(tile_cnt)
    tile_eid = jnp.minimum(
        jnp.searchsorted(cum, jnp.arange(nt, dtype=I32), side="right"),
        E - 1).astype(I32)

    dest = _dest(onehot, starts18, n)
    dest3 = dest.reshape(NWORK, (n // NWORK) // CHUNK, CHUNK)

    ys = _dispatch_scatter(y, dest3, npad)
    hs = _experts(tile_eid, ys, We1, be1, We2, be2, We3, be3, We4, be4, npad)
    hraw = _combine_gather(hs, dest3, n)

    wsig_row = W_sig.reshape(1, ENC)
    bsig11 = b_sig.reshape(1, 1)
    d_dir = W_r1.shape[0] - ENC
    wr1a = W_r1[:ENC]
    wr1b32 = jnp.concatenate(
        [W_r1[ENC:], jnp.zeros((32 - d_dir, 128), F32)], axis=0)
    wr2p = jnp.concatenate([W_r2, jnp.zeros((128, 125), F32)], axis=1)
    br2p = jnp.concatenate([b_r2, jnp.zeros((125,), F32)]).reshape(1, 128)

    sig, rgbp, ssum = _heads(hraw, gtop, vdir2, wsig_row, bsig11,
                             wr1a, wr1b32, b_r1, wr2p, br2p, n)

    sigmas = sig.reshape(nrays, nsamples, 1)
    rgbs = rgbp[:, :3].reshape(nrays, nsamples, 3)
    gates_soft_o = gates.reshape(nrays, nsamples, E)
    gates_hard_o = onehot.reshape(nrays, nsamples, E)
    mean_sigma = (ssum / n).reshape(1)
    num_pts = cnt
    aux_loss = E * jnp.sum((cnt / n) * (gsum.reshape(E) / n))
    return (sigmas, rgbs, gates_soft_o, gates_hard_o,
            mean_sigma, num_pts, aux_loss)
